# baseline (speedup 1.0000x reference)
import jax
import jax.numpy as jnp
from jax.experimental import pallas as pl

N = 10000
E = 160000
M_L = 32
I_LAYERS = 3
N_ITER = 2
ZETA = 1e-08


_SELU_SCALE = 1.0507009873554805
_SELU_ALPHA = 1.6732632423543772


def _selu(x):
    return _SELU_SCALE * jnp.where(x > 0, x, _SELU_ALPHA * (jnp.exp(x) - 1.0))


def _selu_pallas_kernel(x_ref, o_ref):
    x = x_ref[...]
    o_ref[...] = _selu(x)


def _selu_pallas(x):
    rows = x.shape[0]
    blk = 8000
    grid = (rows + blk - 1) // blk
    return pl.pallas_call(
        _selu_pallas_kernel,
        grid=(grid,),
        in_specs=[pl.BlockSpec((blk, x.shape[1]), lambda i: (i, 0))],
        out_specs=pl.BlockSpec((blk, x.shape[1]), lambda i: (i, 0)),
        out_shape=jax.ShapeDtypeStruct(x.shape, x.dtype),
    )(x)


def _compute_net_flows(h, r, src, dst):
    dh = h[src] - h[dst]
    q = jnp.sign(dh) * (jnp.abs(dh) / r + ZETA) ** (1.0 / 1.852)
    d = jax.ops.segment_sum(q, dst, num_segments=N)
    return d, q


def _construct_heads(J, h0, q, r, src, dst, mask):
    hl = r * q * (jnp.abs(q) + ZETA) ** 0.852
    h = h0
    for _ in range(J):
        prop = jax.ops.segment_max(h[src] - hl, dst, num_segments=N)
        prop = jnp.where(jnp.isfinite(prop), prop, h)
        h = jnp.where(mask, h0, jnp.maximum(h, prop))
    return h


def _gnn_layer(g, z, src, dst, We1, We2, Wn1, Wn2):
    m = jnp.concatenate([g[src], g[dst], z], axis=-1)
    z_new = jax.nn.selu(m @ We1) @ We2
    aggr = jax.ops.segment_max(z_new, dst, num_segments=N)
    aggr = jnp.where(jnp.isfinite(aggr), aggr, 0.0)
    g_new = jax.nn.selu(jnp.concatenate([g, aggr], axis=-1) @ Wn1) @ Wn2
    return g_new, z_new


def kernel(x, edge_index, edge_attr, r_iter, W_node_in, W_edge, Wf1, Wf2, Wf3,
           g0_We1, g0_We2, g0_Wn1, g0_Wn2, g1_We1, g1_We2, g1_Wn1, g1_Wn2,
           g2_We1, g2_We2, g2_Wn1, g2_Wn2):
    gcn_ws = [(g0_We1, g0_We2, g0_Wn1, g0_Wn2),
              (g1_We1, g1_We2, g1_Wn1, g1_Wn2),
              (g2_We1, g2_We2, g2_Wn1, g2_Wn2)]
    src = edge_index[0]
    dst = edge_index[1]
    r = edge_attr[:, 0:1]
    h_star = x[:, 0:1]
    d_star = x[:, 1:2]
    mask = h_star != 0
    d_hat, q_hat = _compute_net_flows(h_star, r, src, dst)
    q_tilde = q_hat
    K = N_ITER + jnp.asarray(r_iter, dtype=jnp.int32)
    h_tilde = h_star

    def _body(_, carry):
        d_hat, q_hat, q_tilde, h_tilde = carry
        g = _selu_pallas(jnp.concatenate([d_hat, d_star], axis=-1)) @ W_node_in
        z = _selu_pallas(jnp.concatenate([q_tilde, q_hat], axis=-1)) @ W_edge
        for (We1, We2, Wn1, Wn2) in gcn_ws:
            g, z = _gnn_layer(g, z, src, dst, We1, We2, Wn1, Wn2)
        lat = jax.nn.selu(jnp.concatenate([g[src], g[dst], z], axis=-1))
        dq = jax.nn.selu(jax.nn.selu(lat @ Wf1) @ Wf2) @ Wf3
        q_hat = q_hat + dq
        q_in = q_hat[: E // 2]
        q_hat = jnp.concatenate([q_in, -q_in], axis=0)
        d_hat = jax.ops.segment_sum(q_hat, dst, num_segments=N)
        h_tilde = _construct_heads(I_LAYERS * N_ITER, h_star, q_hat, r, src, dst, mask)
        d_tilde, q_tilde = _compute_net_flows(h_tilde, r, src, dst)
        return (d_hat, q_hat, q_tilde, h_tilde)

    d_hat, q_hat, q_tilde, h_tilde = jax.lax.fori_loop(
        0, K, _body, (d_hat, q_hat, q_tilde, h_tilde))
    return h_tilde


# speedup vs baseline: 7.4460x; 7.4460x over previous
import functools

import jax
import jax.numpy as jnp
from jax import lax
from jax.experimental import pallas as pl
from jax.experimental.pallas import tpu as pltpu
from jax.experimental.pallas import tpu_sc as plsc

N = 10000
E = 160000
EH = E // 2
M = 32
I_LAYERS = 3
N_ITER = 2
ZETA = 1e-08

NC = 2
NS = 16
L = 16
NW = NC * NS

B32 = 320
NP = B32 * NW
B16 = 2 * B32
CW = 992
EP = E + CW
NEG = -3.0e38

_SELU_SCALE = 1.0507009873554805
_SELU_ALPHA = 1.6732632423543772

_mesh = plsc.VectorSubcoreMesh(core_axis_name="c", subcore_axis_name="s")
_CP = pltpu.CompilerParams(use_tc_tiling_on_sc=False, needs_layout_passes=False)


def _selu(x):
    return _SELU_SCALE * jnp.where(x > 0, x, _SELU_ALPHA * (jnp.exp(x) - 1.0))


def _wid():
    return lax.axis_index("s") * NC + lax.axis_index("c")


@functools.partial(
    pl.kernel, mesh=_mesh, compiler_params=_CP,
    out_type=jax.ShapeDtypeStruct((EP,), jnp.int32),
    scratch_types=[pltpu.VMEM((1000,), jnp.int32),
                   pltpu.VMEM((1000,), jnp.int32),
                   pltpu.SemaphoreType.DMA],
)
def _k1_permute(order_hbm, src_hbm, srcs_hbm, ord_v, val_v, sem):
    w = _wid()
    base = w * (E // NW)

    @pl.loop(0, E // NW, step=1000)
    def _(i):
        pltpu.sync_copy(order_hbm.at[pl.ds(base + i, 1000)], ord_v)
        pltpu.async_copy(src_hbm.at[ord_v], val_v, sem).wait()
        pltpu.sync_copy(val_v, srcs_hbm.at[pl.ds(base + i, 1000)])


def _make_k2(n_edges, n_workers):
    per_w = n_edges // n_workers

    @functools.partial(
        pl.kernel, mesh=_mesh, compiler_params=_CP,
        out_type=[jax.ShapeDtypeStruct((n_edges, M), jnp.float32),
                  jax.ShapeDtypeStruct((n_edges, M), jnp.float32)],
        scratch_types=[pltpu.VMEM((1000,), jnp.int32),
                       pltpu.VMEM((1000, M), jnp.float32),
                       pltpu.VMEM_SHARED((NP, M), jnp.float32),
                       pltpu.SemaphoreType.DMA],
    )
    def k2(tab_hbm, ia_hbm, ib_hbm, ga_hbm, gb_hbm, idx_v, rows_v, tab_sh, sem):
        w = _wid()
        sid = lax.axis_index("s")

        @pl.when(sid == 0)
        def _():
            pltpu.sync_copy(tab_hbm, tab_sh)

        plsc.subcore_barrier()

        @pl.when(w < n_workers)
        def _():
            base = w * per_w

            @pl.loop(0, per_w, step=1000)
            def _(i):
                pltpu.sync_copy(ia_hbm.at[pl.ds(base + i, 1000)], idx_v)
                pltpu.async_copy(tab_sh.at[idx_v], rows_v, sem).wait()
                pltpu.sync_copy(rows_v, ga_hbm.at[pl.ds(base + i, 1000)])
                pltpu.sync_copy(ib_hbm.at[pl.ds(base + i, 1000)], idx_v)
                pltpu.async_copy(tab_sh.at[idx_v], rows_v, sem).wait()
                pltpu.sync_copy(rows_v, gb_hbm.at[pl.ds(base + i, 1000)])

    return k2


_k2_full = _make_k2(E, NW)
_k2_half = _make_k2(EH, NS)


@functools.partial(
    pl.kernel, mesh=_mesh, compiler_params=_CP,
    out_type=jax.ShapeDtypeStruct((NW, B32 * M), jnp.float32),
    scratch_types=[pltpu.VMEM((CW,), jnp.int32),
                   pltpu.VMEM((CW,), jnp.int32),
                   pltpu.VMEM((CW, M), jnp.float32),
                   pltpu.VMEM(((B32 + 1) * M,), jnp.float32),
                   pltpu.VMEM((48,), jnp.int32),
                   pltpu.SemaphoreType.DMA],
)
def _k3_segmax(v_hbm, ord_hbm, dsts_hbm, starts_hbm, out_hbm,
               dl_v, ord_v, rows_v, acc_v, st_v, sem):
    w = _wid()
    lo = w * B32
    pltpu.sync_copy(starts_hbm, st_v)
    iota = lax.iota(jnp.int32, L)
    sg = plsc.load_gather(st_v, [jnp.minimum(w + iota, 47)])
    s = sg[0]
    e = sg[1]

    @pl.loop(0, (B32 + 1) * M, step=L)
    def _(i):
        acc_v[pl.ds(i, L)] = jnp.full((L,), NEG, jnp.float32)

    w0 = (s // 8) * 8
    nc = (e - w0 + (CW - 1)) // CW

    @pl.loop(0, nc)
    def _(ci):
        k0 = w0 + ci * CW
        pltpu.sync_copy(dsts_hbm.at[pl.ds(k0, CW)], dl_v)
        pltpu.sync_copy(ord_hbm.at[pl.ds(k0, CW)], ord_v)
        pltpu.async_copy(v_hbm.at[ord_v], rows_v, sem).wait()

        @pl.loop(0, CW, step=L)
        def _(j):
            dl16 = dl_v[pl.ds(j, L)] - lo
            dl16 = jnp.where((dl16 < 0) | (dl16 > B32 - 1), B32, dl16)
            for t in range(L):
                aoff = dl16[t] * M
                jt = j + t
                r0 = rows_v[jt, pl.ds(0, L)]
                r1 = rows_v[jt, pl.ds(L, L)]
                a0 = acc_v[pl.ds(aoff, L)]
                a1 = acc_v[pl.ds(aoff + L, L)]
                acc_v[pl.ds(aoff, L)] = jnp.maximum(a0, r0)
                acc_v[pl.ds(aoff + L, L)] = jnp.maximum(a1, r1)

    @pl.loop(0, B32 * M, step=L)
    def _(i):
        a = acc_v[pl.ds(i, L)]
        acc_v[pl.ds(i, L)] = jnp.where(a < -2.9e38, 0.0, a)

    pltpu.sync_copy(acc_v.at[pl.ds(0, B32 * M)], out_hbm.at[w])


_R4 = E // 125


@functools.partial(
    pl.kernel, mesh=_mesh, compiler_params=_CP,
    out_type=[jax.ShapeDtypeStruct((NC, NP), jnp.float32),
              jax.ShapeDtypeStruct((EP,), jnp.float32)],
    scratch_types=[pltpu.VMEM((40, 125), jnp.int32),
                   pltpu.VMEM((40, 125), jnp.float32),
                   pltpu.VMEM((NP // NS,), jnp.float32),
                   pltpu.VMEM((1000,), jnp.int32),
                   pltpu.VMEM((1000,), jnp.float32),
                   pltpu.VMEM_SHARED((NP,), jnp.float32),
                   pltpu.SemaphoreType.DMA],
)
def _k4_segsum(q2_hbm, d2_hbm, hl_hbm, ord_hbm, dpart_hbm, hls_hbm,
               d2_v, q2_v, z_v, ord_v, val_v, acc_sh, sem):
    cid = lax.axis_index("c")
    sid = lax.axis_index("s")
    w = sid * NC + cid
    stripe = NP // NS

    @pl.loop(0, stripe, step=L)
    def _(i):
        z_v[pl.ds(i, L)] = jnp.zeros((L,), jnp.float32)

    pltpu.sync_copy(z_v, acc_sh.at[pl.ds(sid * stripe, stripe)])
    plsc.subcore_barrier()

    rbase = w * (_R4 // NW)
    pltpu.sync_copy(d2_hbm.at[pl.ds(rbase, 40)], d2_v)
    pltpu.sync_copy(q2_hbm.at[pl.ds(rbase, 40)], q2_v)
    for j in range(40):
        pltpu.sync_copy(q2_v.at[j], acc_sh.at[d2_v.at[j]], add=True)
    plsc.subcore_barrier()

    @pl.when(sid == 0)
    def _():
        pltpu.sync_copy(acc_sh, dpart_hbm.at[cid])

    ebase = w * (E // NW)

    @pl.loop(0, E // NW, step=1000)
    def _(i):
        pltpu.sync_copy(ord_hbm.at[pl.ds(ebase + i, 1000)], ord_v)
        pltpu.async_copy(hl_hbm.at[ord_v], val_v, sem).wait()
        pltpu.sync_copy(val_v, hls_hbm.at[pl.ds(ebase + i, 1000)])


@functools.partial(
    pl.kernel, mesh=_mesh, compiler_params=_CP,
    out_type=[jax.ShapeDtypeStruct((NP,), jnp.float32),
              jax.ShapeDtypeStruct((E,), jnp.float32),
              jax.ShapeDtypeStruct((E,), jnp.float32)],
    scratch_types=[pltpu.VMEM((NP,), jnp.float32),
                   pltpu.VMEM((NP,), jnp.float32),
                   pltpu.VMEM((B16 + 16,), jnp.float32),
                   pltpu.VMEM((B16,), jnp.float32),
                   pltpu.VMEM((CW,), jnp.int32),
                   pltpu.VMEM((CW,), jnp.int32),
                   pltpu.VMEM((CW,), jnp.float32),
                   pltpu.VMEM((2000,), jnp.int32),
                   pltpu.VMEM((2000,), jnp.float32),
                   pltpu.VMEM((48,), jnp.int32),
                   pltpu.VMEM_SHARED((NP,), jnp.float32),
                   pltpu.SemaphoreType.DMA],
)
def _k5_heads(h0_hbm, hls_hbm, srcs_hbm, dsts_hbm, starts_hbm,
              srcn_hbm, dstn_hbm, hout_hbm, hs_hbm, hd_hbm,
              h_ts, h0_ts, acc_v, hn_v, dl_v, sr_v, hl_v,
              in_v, vv_v, st_v, h_sh, sem):
    cid = lax.axis_index("c")
    sid = lax.axis_index("s")

    pltpu.async_copy(h0_hbm, h_ts, sem).wait()
    pltpu.async_copy(h0_hbm, h0_ts, sem).wait()
    pltpu.sync_copy(starts_hbm, st_v)
    iota = lax.iota(jnp.int32, L)
    sg = plsc.load_gather(st_v, [jnp.minimum(2 * sid + iota, 47)])
    s = sg[0]
    e = sg[2]
    lo = sid * B16

    @pl.loop(0, B16 + 16, step=L)
    def _(i):
        acc_v[pl.ds(i, L)] = jnp.full((L,), NEG, jnp.float32)

    w0 = (s // 8) * 8
    nc = (e - w0 + (CW - 1)) // CW

    for _sweep in range(I_LAYERS * N_ITER):
        @pl.loop(0, nc)
        def _(ci):
            k0 = w0 + ci * CW
            pltpu.sync_copy(dsts_hbm.at[pl.ds(k0, CW)], dl_v)
            pltpu.sync_copy(srcs_hbm.at[pl.ds(k0, CW)], sr_v)
            pltpu.sync_copy(hls_hbm.at[pl.ds(k0, CW)], hl_v)

            @pl.loop(0, CW, step=L)
            def _(j):
                dl = dl_v[pl.ds(j, L)] - lo
                dl = jnp.where((dl < 0) | (dl > B16 - 1), B16, dl)
                sv = sr_v[pl.ds(j, L)]
                sv = jnp.minimum(jnp.maximum(sv, 0), NP - 1)
                val = plsc.load_gather(h_ts, [sv]) - hl_v[pl.ds(j, L)]
                for kk in (1, 2, 4, 8):
                    idxk = jnp.maximum(iota - kk, 0)
                    shv = jnp.take(val, idxk)
                    shd = jnp.take(dl, idxk)
                    okm = (shd == dl) & (iota >= kk)
                    val = jnp.where(okm, jnp.maximum(val, shv), val)
                nxt = jnp.take(dl, jnp.minimum(iota + 1, L - 1))
                end = (nxt != dl) | (iota == L - 1)
                cur = plsc.load_gather(acc_v, [dl], mask=end)
                plsc.store_scatter(acc_v, [dl], jnp.maximum(cur, val),
                                   mask=end)

        @pl.loop(0, B16, step=L)
        def _(i):
            a = acc_v[pl.ds(i, L)]
            hold = h_ts[pl.ds(lo + i, L)]
            h0v = h0_ts[pl.ds(lo + i, L)]
            prop = jnp.where(a < -2.9e38, hold, a)
            hn_v[pl.ds(i, L)] = jnp.where(h0v != 0.0, h0v,
                                          jnp.maximum(hold, prop))
            acc_v[pl.ds(i, L)] = jnp.full((L,), NEG, jnp.float32)

        pltpu.sync_copy(hn_v, h_sh.at[pl.ds(lo, B16)])
        plsc.subcore_barrier()
        pltpu.sync_copy(h_sh, h_ts)
        plsc.subcore_barrier()

    @pl.when((cid == 0) & (sid == 0))
    def _():
        pltpu.sync_copy(h_ts, hout_hbm)

    nbase = sid * (E // NS)
    idxsrc_hbm = [srcn_hbm, dstn_hbm]
    out_hbms = [hs_hbm, hd_hbm]
    for side in range(2):
        @pl.when(cid == side)
        def _():
            @pl.loop(0, E // NS, step=2000)
            def _(i):
                pltpu.sync_copy(idxsrc_hbm[side].at[pl.ds(nbase + i, 2000)],
                                in_v)

                @pl.loop(0, 2000, step=L)
                def _(j):
                    vv_v[pl.ds(j, L)] = plsc.load_gather(
                        h_ts, [in_v[pl.ds(j, L)]])

                pltpu.sync_copy(vv_v, out_hbms[side].at[pl.ds(nbase + i, 2000)])


@functools.partial(
    pl.kernel, mesh=_mesh, compiler_params=_CP,
    out_type=[jax.ShapeDtypeStruct((E,), jnp.float32),
              jax.ShapeDtypeStruct((E,), jnp.float32)],
    scratch_types=[pltpu.VMEM((1000,), jnp.int32),
                   pltpu.VMEM((1000,), jnp.float32),
                   pltpu.SemaphoreType.DMA],
)
def _k6_gather_hpair(h_hbm, src_hbm, dst_hbm, hs_hbm, hd_hbm, idx_v, val_v, sem):
    w = _wid()
    base = w * (E // NW)

    @pl.loop(0, E // NW, step=1000)
    def _(i):
        pltpu.sync_copy(src_hbm.at[pl.ds(base + i, 1000)], idx_v)
        pltpu.async_copy(h_hbm.at[idx_v], val_v, sem).wait()
        pltpu.sync_copy(val_v, hs_hbm.at[pl.ds(base + i, 1000)])
        pltpu.sync_copy(dst_hbm.at[pl.ds(base + i, 1000)], idx_v)
        pltpu.async_copy(h_hbm.at[idx_v], val_v, sem).wait()
        pltpu.sync_copy(val_v, hd_hbm.at[pl.ds(base + i, 1000)])


def _tc_g_kernel(d0_ref, d1_ref, ds_ref, w0_ref, w1_ref, o_ref):
    dh = d0_ref[...] + d1_ref[...]
    s0 = _selu(dh)
    s1 = _selu(ds_ref[...])
    o_ref[...] = (s0[:, None] * w0_ref[...][None, :]
                  + s1[:, None] * w1_ref[...][None, :])


def _tc_g(d0, d1, dsp, w):
    return pl.pallas_call(
        _tc_g_kernel,
        out_shape=jax.ShapeDtypeStruct((NP, M), jnp.float32),
    )(d0, d1, dsp, w[0], w[1])


def _tc_z_kernel(qt_ref, qh_ref, w0_ref, w1_ref, o_ref):
    blk = 6400
    pid = pl.program_id(0)
    s0 = _selu(qt_ref[pl.ds(pid * blk, blk)])
    s1 = _selu(qh_ref[pl.ds(pid * blk, blk)])
    o_ref[...] = (s0[:, None] * w0_ref[...][None, :]
                  + s1[:, None] * w1_ref[...][None, :])


def _tc_z(qt, qh, w):
    blk = 6400
    return pl.pallas_call(
        _tc_z_kernel,
        grid=(E // blk,),
        in_specs=[pl.BlockSpec((E,), lambda i: (0,)),
                  pl.BlockSpec((E,), lambda i: (0,)),
                  pl.BlockSpec((M,), lambda i: (0,)),
                  pl.BlockSpec((M,), lambda i: (0,))],
        out_specs=pl.BlockSpec((blk, M), lambda i: (i, 0)),
        out_shape=jax.ShapeDtypeStruct((E, M), jnp.float32),
    )(qt, qh, w[0], w[1])


def _dot(a, b):
    return jnp.dot(a.astype(jnp.bfloat16), b.astype(jnp.bfloat16),
                   preferred_element_type=jnp.float32)


def _tc_znew_kernel(gs_ref, gd_ref, z_ref, a_ref, b_ref, c_ref, w2_ref, o_ref):
    t = (_dot(gs_ref[...], a_ref[...]) + _dot(gd_ref[...], b_ref[...])
         + _dot(z_ref[...], c_ref[...]))
    o_ref[...] = _dot(_selu(t), w2_ref[...])


def _tc_znew(gs, gd, z, wa, wb, wc, w2):
    blk = 4000
    wspec = pl.BlockSpec((M, M), lambda i: (0, 0))
    return pl.pallas_call(
        _tc_znew_kernel,
        grid=(E // blk,),
        in_specs=[pl.BlockSpec((blk, M), lambda i: (i, 0))] * 3
                 + [wspec] * 4,
        out_specs=pl.BlockSpec((blk, M), lambda i: (i, 0)),
        out_shape=jax.ShapeDtypeStruct((E, M), jnp.float32),
    )(gs, gd, z, wa, wb, wc, w2)


def _tc_gnew_kernel(g_ref, ag_ref, wt_ref, wb_ref, w2_ref, o_ref):
    u = _dot(g_ref[...], wt_ref[...]) + _dot(ag_ref[...], wb_ref[...])
    o_ref[...] = _dot(_selu(u), w2_ref[...])


def _tc_gnew(g, aggr, wt, wb, w2):
    blk = 2560
    wspec = pl.BlockSpec((M, M), lambda i: (0, 0))
    return pl.pallas_call(
        _tc_gnew_kernel,
        grid=(NP // blk,),
        in_specs=[pl.BlockSpec((blk, M), lambda i: (i, 0))] * 2 + [wspec] * 3,
        out_specs=pl.BlockSpec((blk, M), lambda i: (i, 0)),
        out_shape=jax.ShapeDtypeStruct((NP, M), jnp.float32),
    )(g, aggr, wt, wb, w2)


def _tc_dq_kernel(gs_ref, gd_ref, z_ref, qh_ref, r_ref,
                  wa_ref, wb_ref, wc_ref, w2_ref, w3_ref, qn_ref, hl_ref):
    blk = 3200
    pid = pl.program_id(0)
    t = (_dot(_selu(gs_ref[...]), wa_ref[...])
         + _dot(_selu(gd_ref[...]), wb_ref[...])
         + _dot(_selu(z_ref[...]), wc_ref[...]))
    t = _dot(_selu(t), w2_ref[...])
    t = _selu(t)
    tb = t.astype(jnp.bfloat16).astype(jnp.float32)
    w3b = w3_ref[...].astype(jnp.bfloat16).astype(jnp.float32)
    dq = jnp.sum(tb * w3b[None, :], axis=1)
    qn = qh_ref[pl.ds(pid * blk, blk)] + dq
    qn_ref[pl.ds(pid * blk, blk)] = qn
    ap = jnp.abs(qn) + ZETA
    hl = r_ref[pl.ds(pid * blk, blk)] * qn * jnp.exp(0.852 * jnp.log(ap))
    hl_ref[pl.ds(pid * blk, blk)] = hl


def _tc_dq(gs, gd, z, qh, r, wa, wb, wc, w2, w3row):
    blk = 3200
    wspec = pl.BlockSpec((M, M), lambda i: (0, 0))
    espec = pl.BlockSpec((EH,), lambda i: (0,))
    return pl.pallas_call(
        _tc_dq_kernel,
        grid=(EH // blk,),
        in_specs=[pl.BlockSpec((blk, M), lambda i: (i, 0)),
                  pl.BlockSpec((blk, M), lambda i: (i, 0)),
                  pl.BlockSpec((blk, M), lambda i: (i, 0)),
                  espec, espec,
                  wspec, wspec, wspec, wspec,
                  pl.BlockSpec((M,), lambda i: (0,))],
        out_specs=[espec, espec],
        out_shape=[jax.ShapeDtypeStruct((EH,), jnp.float32),
                   jax.ShapeDtypeStruct((EH,), jnp.float32)],
    )(gs, gd, z, qh, r, wa, wb, wc, w2, w3row)


def _tc_qflow_kernel(hs_ref, hd_ref, r_ref, o_ref):
    dh = hs_ref[...] - hd_ref[...]
    base = jnp.abs(dh) / r_ref[...] + ZETA
    o_ref[...] = jnp.sign(dh) * jnp.exp((1.0 / 1.852) * jnp.log(base))


def _tc_qflow(hs, hd, r):
    return pl.pallas_call(
        _tc_qflow_kernel,
        out_shape=jax.ShapeDtypeStruct((E,), jnp.float32),
    )(hs, hd, r)


def kernel(x, edge_index, edge_attr, r_iter, W_node_in, W_edge, Wf1, Wf2, Wf3,
           g0_We1, g0_We2, g0_Wn1, g0_Wn2, g1_We1, g1_We2, g1_Wn1, g1_Wn2,
           g2_We1, g2_We2, g2_Wn1, g2_Wn2):
    gcn_ws = [(g0_We1, g0_We2, g0_Wn1, g0_Wn2),
              (g1_We1, g1_We2, g1_Wn1, g1_Wn2),
              (g2_We1, g2_We2, g2_Wn1, g2_Wn2)]

    src = edge_index[0].astype(jnp.int32)
    dst = edge_index[1].astype(jnp.int32)
    r_nat = edge_attr[:, 0]
    h_star = x[:, 0]
    d_star = x[:, 1]
    h0p = jnp.concatenate([h_star, jnp.zeros((NP - N,), jnp.float32)])
    dsp = jnp.concatenate([d_star, jnp.zeros((NP - N,), jnp.float32)])

    dsts, order = lax.sort([dst, jnp.arange(E, dtype=jnp.int32)], num_keys=1)
    bounds = jnp.arange(40, dtype=jnp.int32) * B32
    starts = jnp.searchsorted(dsts, bounds[:33]).astype(jnp.int32)
    starts = jnp.concatenate([starts, jnp.full((15,), E, jnp.int32)])
    dsts_p = jnp.concatenate([dsts, jnp.full((CW,), 1 << 29, jnp.int32)])
    order_p = jnp.concatenate([order, jnp.zeros((CW,), jnp.int32)])
    d2 = dst.reshape(_R4, 125)

    srcs_p = _k1_permute(order_p, src)

    w3row = Wf3[:, 0]
    r_half = r_nat[:EH]
    wf1a, wf1b, wf1c = Wf1[:M], Wf1[M:2 * M], Wf1[2 * M:]

    hs0, hd0 = _k6_gather_hpair(h0p, src, dst)
    q0 = _tc_qflow(hs0, hd0, r_nat)
    dpart0, _unused_hls = _k4_segsum(q0.reshape(_R4, 125), d2, r_nat, order_p)

    K = N_ITER + jnp.asarray(r_iter, dtype=jnp.int32)

    def _body(_, carry):
        dpart, q_hat, q_tilde, hpad = carry
        g = _tc_g(dpart[0], dpart[1], dsp, W_node_in)
        z = _tc_z(q_tilde, q_hat, W_edge)
        for (We1, We2, Wn1, Wn2) in gcn_ws:
            gs, gd = _k2_full(g, src, dst)
            z = _tc_znew(gs, gd, z, We1[:M], We1[M:2 * M], We1[2 * M:], We2)
            aggr = _k3_segmax(z, order_p, dsts_p, starts)
            aggr = aggr.reshape(NP, M)
            g = _tc_gnew(g, aggr, Wn1[:M], Wn1[M:], Wn2)
        gs, gd = _k2_half(g, src, dst)
        qn, hlh = _tc_dq(gs, gd, z, q_hat[:EH], r_half,
                         wf1a, wf1b, wf1c, Wf2, w3row)
        q_hat = jnp.concatenate([qn, -qn])
        hl = jnp.concatenate([hlh, -hlh])
        dpart, hls = _k4_segsum(q_hat.reshape(_R4, 125), d2, hl, order_p)
        hpad, hs, hd = _k5_heads(h0p, hls, srcs_p, dsts_p, starts, src, dst)
        q_tilde = _tc_qflow(hs, hd, r_nat)
        return (dpart, q_hat, q_tilde, hpad)

    carry = (dpart0, q0, q0, h0p)
    dpart, q_hat, q_tilde, hpad = lax.fori_loop(0, K, _body, carry)
    return hpad[:N].reshape(N, 1)
